# initial kernel scaffold (unmeasured)
import jax
import jax.numpy as jnp
from jax import lax
from jax.experimental import pallas as pl
from jax.experimental.pallas import tpu as pltpu

N_DEV = 32


def kernel(Q, K, V):
    b, s_loc, h, d = Q.shape
    hd = h * d
    n_hops = N_DEV - 1
    scale = d ** -0.5

    Qr = Q.reshape(b, s_loc, hd)
    Kr = K.reshape(b, s_loc, hd)
    Vr = V.reshape(b, s_loc, hd)

    def body(q_ref, k_ref, v_ref, out_ref, kfull, vfull,
             k_send, k_recv, v_send, v_recv):
        my = lax.axis_index("i")
        right = lax.rem(my + 1, N_DEV)
        left = lax.rem(my + N_DEV - 1, N_DEV)

        barrier_sem = pltpu.get_barrier_semaphore()
        for nbr in (left, right):
            pl.semaphore_signal(
                barrier_sem, inc=1,
                device_id=(nbr,), device_id_type=pl.DeviceIdType.MESH,
            )
        pl.semaphore_wait(barrier_sem, 2)

        kfull[my] = k_ref[...].astype(jnp.bfloat16)
        vfull[my] = v_ref[...].astype(jnp.bfloat16)

        for t in range(n_hops):
            k_org = lax.rem(my - t + N_DEV, N_DEV)
            v_org = lax.rem(my + t, N_DEV)
            k_rdma = pltpu.make_async_remote_copy(
                src_ref=kfull.at[k_org], dst_ref=kfull.at[k_org],
                send_sem=k_send.at[t], recv_sem=k_recv.at[t],
                device_id=(right,), device_id_type=pl.DeviceIdType.MESH,
            )
            v_rdma = pltpu.make_async_remote_copy(
                src_ref=vfull.at[v_org], dst_ref=vfull.at[v_org],
                send_sem=v_send.at[t], recv_sem=v_recv.at[t],
                device_id=(left,), device_id_type=pl.DeviceIdType.MESH,
            )
            k_rdma.start()
            v_rdma.start()
            k_rdma.wait()
            v_rdma.wait()

        for bb in range(b):
            for hh in range(h):
                q = q_ref[bb, :, hh * d:(hh + 1) * d].astype(jnp.bfloat16)
                k = kfull[:, bb, :, hh * d:(hh + 1) * d].reshape(
                    N_DEV * s_loc, d)
                s = lax.dot_general(
                    q, k, (((1,), (1,)), ((), ())),
                    preferred_element_type=jnp.float32,
                ) * scale
                m = jnp.max(s, axis=-1, keepdims=True)
                p = jnp.exp(s - m)
                l = jnp.sum(p, axis=-1, keepdims=True)
                p = (p / l).astype(jnp.bfloat16)
                v = vfull[:, bb, :, hh * d:(hh + 1) * d].reshape(
                    N_DEV * s_loc, d)
                o = lax.dot_general(
                    p, v, (((1,), (0,)), ((), ())),
                    preferred_element_type=jnp.float32,
                )
                out_ref[bb, :, hh * d:(hh + 1) * d] = o

    out = pl.pallas_call(
        body,
        out_shape=jax.ShapeDtypeStruct((b, s_loc, hd), jnp.float32),
        in_specs=[pl.BlockSpec(memory_space=pltpu.VMEM)] * 3,
        out_specs=pl.BlockSpec(memory_space=pltpu.VMEM),
        scratch_shapes=[
            pltpu.VMEM((N_DEV, b, s_loc, hd), jnp.bfloat16),
            pltpu.VMEM((N_DEV, b, s_loc, hd), jnp.bfloat16),
            pltpu.SemaphoreType.DMA((n_hops,)),
            pltpu.SemaphoreType.DMA((n_hops,)),
            pltpu.SemaphoreType.DMA((n_hops,)),
            pltpu.SemaphoreType.DMA((n_hops,)),
        ],
        compiler_params=pltpu.CompilerParams(collective_id=0),
    )(Qr, Kr, Vr)
    return out.reshape(b, s_loc, h, d)


# baseline (device time: 477272 ns/iter reference)
import jax
import jax.numpy as jnp
from jax import lax
from jax.experimental import pallas as pl
from jax.experimental.pallas import tpu as pltpu

N_DEV = 32


def kernel(Q, K, V):
    b, s_loc, h, d = Q.shape
    hd = h * d
    n_hops = N_DEV - 1
    scale = d ** -0.5

    Qr = Q.reshape(b, s_loc, hd)
    Kr = K.reshape(b, s_loc, hd)
    Vr = V.reshape(b, s_loc, hd)

    def body(q_ref, k_ref, v_ref, out_ref, kfull, vfull,
             k_send, k_recv, v_send, v_recv):
        my = lax.axis_index("i")
        right = lax.rem(my + 1, N_DEV)
        left = lax.rem(my + N_DEV - 1, N_DEV)

        barrier_sem = pltpu.get_barrier_semaphore()
        for nbr in (left, right):
            pl.semaphore_signal(
                barrier_sem, inc=1,
                device_id=(nbr,), device_id_type=pl.DeviceIdType.MESH,
            )
        pl.semaphore_wait(barrier_sem, 2)

        kfull[my] = k_ref[...].astype(jnp.bfloat16)
        vfull[my] = v_ref[...].astype(jnp.bfloat16)

        for t in range(n_hops):
            k_org = lax.rem(my - t + N_DEV, N_DEV)
            v_org = lax.rem(my + t, N_DEV)
            k_rdma = pltpu.make_async_remote_copy(
                src_ref=kfull.at[k_org], dst_ref=kfull.at[k_org],
                send_sem=k_send.at[t], recv_sem=k_recv.at[t],
                device_id=(right,), device_id_type=pl.DeviceIdType.MESH,
            )
            v_rdma = pltpu.make_async_remote_copy(
                src_ref=vfull.at[v_org], dst_ref=vfull.at[v_org],
                send_sem=v_send.at[t], recv_sem=v_recv.at[t],
                device_id=(left,), device_id_type=pl.DeviceIdType.MESH,
            )
            k_rdma.start()
            v_rdma.start()
            k_rdma.wait()
            v_rdma.wait()

        for bb in range(b):
            for hh in range(h):
                q = q_ref[bb, :, hh * d:(hh + 1) * d].astype(jnp.bfloat16)
                k = kfull[:, bb, :, hh * d:(hh + 1) * d].reshape(
                    N_DEV * s_loc, d)
                s = lax.dot_general(
                    q, k, (((1,), (1,)), ((), ())),
                    preferred_element_type=jnp.float32,
                ) * scale
                m = jnp.max(s, axis=-1, keepdims=True)
                p = jnp.exp(s - m)
                l = jnp.sum(p, axis=-1, keepdims=True)
                p = (p / l).astype(jnp.bfloat16)
                v = vfull[:, bb, :, hh * d:(hh + 1) * d].reshape(
                    N_DEV * s_loc, d)
                o = lax.dot_general(
                    p, v, (((1,), (0,)), ((), ())),
                    preferred_element_type=jnp.float32,
                )
                out_ref[bb, :, hh * d:(hh + 1) * d] = o

    out = pl.pallas_call(
        body,
        out_shape=jax.ShapeDtypeStruct((b, s_loc, hd), jnp.float32),
        in_specs=[pl.BlockSpec(memory_space=pltpu.VMEM)] * 3,
        out_specs=pl.BlockSpec(memory_space=pltpu.VMEM),
        scratch_shapes=[
            pltpu.VMEM((N_DEV, b, s_loc, hd), jnp.bfloat16),
            pltpu.VMEM((N_DEV, b, s_loc, hd), jnp.bfloat16),
            pltpu.SemaphoreType.DMA((n_hops,)),
            pltpu.SemaphoreType.DMA((n_hops,)),
            pltpu.SemaphoreType.DMA((n_hops,)),
            pltpu.SemaphoreType.DMA((n_hops,)),
        ],
        compiler_params=pltpu.CompilerParams(
            collective_id=0, vmem_limit_bytes=100 * 1024 * 1024,
        ),
    )(Qr, Kr, Vr)
    return out.reshape(b, s_loc, h, d)


# device time: 411564 ns/iter; 1.1597x vs baseline; 1.1597x over previous
import jax
import jax.numpy as jnp
from jax import lax
from jax.experimental import pallas as pl
from jax.experimental.pallas import tpu as pltpu

N_DEV = 32


def kernel(Q, K, V):
    b, s_loc, h, d = Q.shape
    hd = h * d
    n_hops = N_DEV - 1
    scale = d ** -0.5

    Qr = Q.reshape(b, s_loc, hd)
    Kr = K.reshape(b, s_loc, hd)
    Vr = V.reshape(b, s_loc, hd)

    def body(q_ref, k_ref, v_ref, out_ref, kfull, vfull,
             k_send, k_recv, v_send, v_recv):
        my = lax.axis_index("i")
        right = lax.rem(my + 1, N_DEV)
        left = lax.rem(my + N_DEV - 1, N_DEV)

        barrier_sem = pltpu.get_barrier_semaphore()
        for nbr in (left, right):
            pl.semaphore_signal(
                barrier_sem, inc=1,
                device_id=(nbr,), device_id_type=pl.DeviceIdType.MESH,
            )
        pl.semaphore_wait(barrier_sem, 2)

        kfull[my] = k_ref[...].astype(jnp.bfloat16)
        vfull[my] = v_ref[...].astype(jnp.bfloat16)

        for t in range(n_hops):
            k_org = lax.rem(my - t + N_DEV, N_DEV)
            v_org = lax.rem(my + t, N_DEV)
            k_rdma = pltpu.make_async_remote_copy(
                src_ref=kfull.at[k_org], dst_ref=kfull.at[k_org],
                send_sem=k_send.at[t], recv_sem=k_recv.at[t],
                device_id=(right,), device_id_type=pl.DeviceIdType.MESH,
            )
            v_rdma = pltpu.make_async_remote_copy(
                src_ref=vfull.at[v_org], dst_ref=vfull.at[v_org],
                send_sem=v_send.at[t], recv_sem=v_recv.at[t],
                device_id=(left,), device_id_type=pl.DeviceIdType.MESH,
            )
            k_rdma.start()
            v_rdma.start()
            k_rdma.wait()
            v_rdma.wait()

        import os
        if os.environ.get("RING_ONLY"):
            out_ref[...] = jnp.zeros((b, s_loc, hd), jnp.float32)
            return
        for bb in range(b):
            for hh in range(h):
                q = q_ref[bb, :, hh * d:(hh + 1) * d].astype(jnp.bfloat16)
                k = kfull[:, bb, :, hh * d:(hh + 1) * d].reshape(
                    N_DEV * s_loc, d)
                s = lax.dot_general(
                    q, k, (((1,), (1,)), ((), ())),
                    preferred_element_type=jnp.float32,
                ) * scale
                m = jnp.max(s, axis=-1, keepdims=True)
                p = jnp.exp(s - m)
                l = jnp.sum(p, axis=-1, keepdims=True)
                p = (p / l).astype(jnp.bfloat16)
                v = vfull[:, bb, :, hh * d:(hh + 1) * d].reshape(
                    N_DEV * s_loc, d)
                o = lax.dot_general(
                    p, v, (((1,), (0,)), ((), ())),
                    preferred_element_type=jnp.float32,
                )
                out_ref[bb, :, hh * d:(hh + 1) * d] = o

    out = pl.pallas_call(
        body,
        out_shape=jax.ShapeDtypeStruct((b, s_loc, hd), jnp.float32),
        in_specs=[pl.BlockSpec(memory_space=pltpu.VMEM)] * 3,
        out_specs=pl.BlockSpec(memory_space=pltpu.VMEM),
        scratch_shapes=[
            pltpu.VMEM((N_DEV, b, s_loc, hd), jnp.bfloat16),
            pltpu.VMEM((N_DEV, b, s_loc, hd), jnp.bfloat16),
            pltpu.SemaphoreType.DMA((n_hops,)),
            pltpu.SemaphoreType.DMA((n_hops,)),
            pltpu.SemaphoreType.DMA((n_hops,)),
            pltpu.SemaphoreType.DMA((n_hops,)),
        ],
        compiler_params=pltpu.CompilerParams(
            collective_id=0, vmem_limit_bytes=100 * 1024 * 1024,
        ),
    )(Qr, Kr, Vr)
    return out.reshape(b, s_loc, h, d)
